# baseline (speedup 1.0000x reference)
import jax
import jax.numpy as jnp
from jax import lax
from jax.experimental import pallas as pl
from jax.experimental.pallas import tpu as pltpu
from jax.experimental.pallas import tpu_sc as plsc

NUM_CORES = 2
NUM_SUBCORES = 16
LANES = 16
NW = NUM_CORES * NUM_SUBCORES

BATCH = 16384
EMBED_DIM = 32
ROW_W = 128
PACK = ROW_W // EMBED_DIM
B_PER_W = BATCH // NW
HALF = B_PER_W // 2


def _mf_kernel(user_ids, item_ids, user_rows, item_rows, user_bias,
               item_bias, global_bias, out_hbm,
               uidx_v, iidx_v, urow4_v, irow4_v, ucol_v, icol_v,
               urows_v, irows_v, ub_v, ib_v, gb_v, out_v, sem):
    wid = lax.axis_index("s") * NUM_CORES + lax.axis_index("c")
    base = wid * B_PER_W

    pltpu.sync_copy(user_ids.at[pl.ds(base, B_PER_W)], uidx_v)
    pltpu.sync_copy(item_ids.at[pl.ds(base, B_PER_W)], iidx_v)
    pltpu.sync_copy(global_bias, gb_v)

    cub = pltpu.async_copy(user_bias.at[uidx_v], ub_v, sem)
    cib = pltpu.async_copy(item_bias.at[iidx_v], ib_v, sem)

    def split_body(blk, carry):
        off = blk * LANES
        u = uidx_v[pl.ds(off, LANES)]
        i = iidx_v[pl.ds(off, LANES)]
        urow4_v[pl.ds(off, LANES)] = lax.shift_right_logical(u, 2)
        irow4_v[pl.ds(off, LANES)] = lax.shift_right_logical(i, 2)
        ucol_v[pl.ds(off, LANES)] = (u & 3) * EMBED_DIM
        icol_v[pl.ds(off, LANES)] = (i & 3) * EMBED_DIM
        return carry

    lax.fori_loop(0, B_PER_W // LANES, split_body, 0, unroll=4)

    cub.wait()
    cib.wait()
    gb = gb_v[...]

    for h in range(2):
        hoff = h * HALF
        cu = pltpu.async_copy(
            user_rows.at[urow4_v.at[pl.ds(hoff, HALF)]], urows_v, sem)
        ci = pltpu.async_copy(
            item_rows.at[irow4_v.at[pl.ds(hoff, HALF)]], irows_v, sem)
        cu.wait()
        ci.wait()

        def block_body(blk, carry):
            off = hoff + blk * LANES
            rows = blk * LANES + lax.iota(jnp.int32, LANES)
            ucols = ucol_v[pl.ds(off, LANES)]
            icols = icol_v[pl.ds(off, LANES)]
            acc = ub_v[pl.ds(off, LANES)] + ib_v[pl.ds(off, LANES)] + gb
            for d in range(EMBED_DIM):
                u = plsc.load_gather(urows_v, [rows, ucols + d])
                v = plsc.load_gather(irows_v, [rows, icols + d])
                acc = acc + u * v
            out_v[pl.ds(off, LANES)] = acc
            return carry

        lax.fori_loop(0, HALF // LANES, block_body, 0, unroll=2)

    pltpu.sync_copy(out_v, out_hbm.at[pl.ds(base, B_PER_W)])


@jax.jit
def kernel(user_ids, item_ids, user_table, item_table, user_bias, item_bias,
           global_bias):
    mesh = plsc.VectorSubcoreMesh(core_axis_name="c", subcore_axis_name="s")
    run = pl.kernel(
        _mf_kernel, mesh=mesh,
        compiler_params=pltpu.CompilerParams(
            needs_layout_passes=False, use_tc_tiling_on_sc=False),
        out_type=jax.ShapeDtypeStruct((BATCH,), jnp.float32),
        scratch_types=[
            pltpu.VMEM((B_PER_W,), jnp.int32),
            pltpu.VMEM((B_PER_W,), jnp.int32),
            pltpu.VMEM((B_PER_W,), jnp.int32),
            pltpu.VMEM((B_PER_W,), jnp.int32),
            pltpu.VMEM((B_PER_W,), jnp.int32),
            pltpu.VMEM((B_PER_W,), jnp.int32),
            pltpu.VMEM((HALF, ROW_W), jnp.float32),
            pltpu.VMEM((HALF, ROW_W), jnp.float32),
            pltpu.VMEM((B_PER_W,), jnp.float32),
            pltpu.VMEM((B_PER_W,), jnp.float32),
            pltpu.VMEM((LANES,), jnp.float32),
            pltpu.VMEM((B_PER_W,), jnp.float32),
            pltpu.SemaphoreType.DMA,
        ],
    )
    n4 = user_table.shape[0] * EMBED_DIM // ROW_W
    gb16 = jnp.broadcast_to(global_bias.astype(jnp.float32), (LANES,))
    return run(user_ids.astype(jnp.int32), item_ids.astype(jnp.int32),
               user_table.reshape(n4, ROW_W), item_table.reshape(n4, ROW_W),
               user_bias.reshape(-1), item_bias.reshape(-1), gb16)
